# baseline (speedup 1.0000x reference)
import jax
import jax.numpy as jnp
from jax import lax
from jax.experimental import pallas as pl
from jax.experimental.pallas import tpu as pltpu
from jax.experimental.pallas import tpu_sc as plsc

_NC = 2
_NS = 16
_NW = _NC * _NS
_L = 16
_CH = 2048
_CHV = _CH // _L
_D = 10


def _sc_body(x_hbm, bins_hbm, out_hbm,
             xb0, xb1, pb0, pb1, ib0, ib1, binsb, zb,
             si0, si1, so0, so1):
    M = x_hbm.shape[0]
    rows_per_w = M // _NW
    wid = lax.axis_index("s") * _NC + lax.axis_index("c")
    rbase = wid * rows_per_w

    pltpu.sync_copy(bins_hbm, binsb.at[pl.ds(0, 10)])
    vb = binsb[pl.ds(0, _L)]
    thr = [jnp.broadcast_to(vb[k], (_L,)) for k in range(1, 9)]
    vone = jnp.full((_L,), 1.0, jnp.float32)
    vzero = jnp.zeros((_L,), jnp.float32)
    viota = lax.iota(jnp.int32, _L)

    @plsc.parallel_loop(0, _CHV, 1, unroll=8)
    def _(v):
        pb0[0, pl.ds(v * _L, _L)] = vone
        pb1[0, pl.ds(v * _L, _L)] = vone

    @plsc.parallel_loop(0, 9 * _CHV, 1, unroll=8)
    def _(v):
        c = v // _CHV
        p = (v % _CHV) * _L
        pb0[c + 1, pl.ds(p, _L)] = vzero
        pb1[c + 1, pl.ds(p, _L)] = vzero

    vo = pb0[0, pl.ds(0, _L)]
    zb[pl.ds(0, _L)] = vzero
    vz = zb[pl.ds(0, _L)]

    def _in_copy(xb, si, r, off):
        return pltpu.make_async_copy(x_hbm.at[r, pl.ds(off, _CH)], xb, si)

    def _out_copy(pbk, so, r, off):
        return pltpu.make_async_copy(
            pbk, out_hbm.at[pl.ds(0, _D), r, pl.ds(off, _CH)], so)

    _in_copy(xb0, si0, rbase, 0).start()

    def _bucket(xv):
        acc = jnp.where(xv > thr[0], 1, 0)
        for th in thr[1:]:
            acc = acc + jnp.where(xv > th, 1, 0)
        return acc

    N = x_hbm.shape[1]
    cpr = N // _CH
    nchunks = rows_per_w * cpr
    pairs = nchunks // 2

    def _rc(t):
        return rbase + t // cpr, (t % cpr) * _CH

    def _chunk(t2, parity, xb, pbk, ibk, si, so):
        t = 2 * t2 + parity
        r, off = _rc(t)

        tn = t + 1
        if parity == 0:
            rn, offn = _rc(tn)
            _in_copy(xb1, si1, rn, offn).start()
        else:
            @pl.when(tn < nchunks)
            def _():
                rn, offn = _rc(tn)
                _in_copy(xb0, si0, rn, offn).start()

        @pl.when(t2 > 0)
        def _():
            _out_copy(pbk, so, r, off).wait()

        _in_copy(xb, si, r, off).wait()

        @pl.when(t2 > 0)
        def _():
            @plsc.parallel_loop(0, _CHV, 1, unroll=8)
            def _(v):
                old = ibk[pl.ds(v * _L, _L)]
                xv = xb[pl.ds(v * _L, _L)]
                new = _bucket(xv) + 1
                posv = v * _L + viota
                ibk[pl.ds(v * _L, _L)] = new
                plsc.store_scatter(pbk, [new, posv], vo)
                plsc.store_scatter(pbk, [old, posv], vz, mask=old != new)

        @pl.when(t2 == 0)
        def _():
            @plsc.parallel_loop(0, _CHV, 1, unroll=8)
            def _(v):
                xv = xb[pl.ds(v * _L, _L)]
                new = _bucket(xv) + 1
                posv = v * _L + viota
                ibk[pl.ds(v * _L, _L)] = new
                plsc.store_scatter(pbk, [new, posv], vo)

        _out_copy(pbk, so, r, off).start()

    def _pair(t2, _):
        _chunk(t2, 0, xb0, pb0, ib0, si0, so0)
        _chunk(t2, 1, xb1, pb1, ib1, si1, so1)
        return 0
    lax.fori_loop(0, pairs, _pair, 0)

    r, off = _rc(nchunks - 2)
    _out_copy(pb0, so0, r, off).wait()
    r, off = _rc(nchunks - 1)
    _out_copy(pb1, so1, r, off).wait()


def kernel(x, bins):
    M, N = x.shape
    mesh = plsc.VectorSubcoreMesh(core_axis_name="c", subcore_axis_name="s")
    run = pl.kernel(
        _sc_body,
        out_type=jax.ShapeDtypeStruct((_D, M, N), jnp.float32),
        mesh=mesh,
        scratch_types=[
            pltpu.VMEM((_CH,), jnp.float32),
            pltpu.VMEM((_CH,), jnp.float32),
            pltpu.VMEM((_D, _CH), jnp.float32),
            pltpu.VMEM((_D, _CH), jnp.float32),
            pltpu.VMEM((_CH,), jnp.int32),
            pltpu.VMEM((_CH,), jnp.int32),
            pltpu.VMEM((16,), jnp.float32),
            pltpu.VMEM((16,), jnp.float32),
            pltpu.SemaphoreType.DMA,
            pltpu.SemaphoreType.DMA,
            pltpu.SemaphoreType.DMA,
            pltpu.SemaphoreType.DMA,
        ],
        compiler_params=pltpu.CompilerParams(needs_layout_passes=False),
    )
    out = run(x, bins)
    return jnp.transpose(out, (1, 2, 0))


# speedup vs baseline: 1.1731x; 1.1731x over previous
import jax
import jax.numpy as jnp
from jax import lax
from jax.experimental import pallas as pl
from jax.experimental.pallas import tpu as pltpu
from jax.experimental.pallas import tpu_sc as plsc

_NC = 2
_NS = 16
_NW = _NC * _NS
_L = 16
_CH = 4096
_CHV = _CH // _L
_D = 10


def _sc_body(x_hbm, bins_hbm, out_hbm,
             xb0, xb1, pb0, pb1, ib0, ib1, onesb, binsb, zb,
             si0, si1, so0, so1):
    M = x_hbm.shape[0]
    rows_per_w = M // _NW
    wid = lax.axis_index("s") * _NC + lax.axis_index("c")
    rbase = wid * rows_per_w

    pltpu.sync_copy(bins_hbm, binsb.at[pl.ds(0, 10)])
    vb = binsb[pl.ds(0, _L)]
    thr = [jnp.broadcast_to(vb[k], (_L,)) for k in range(1, 9)]
    vone = jnp.full((_L,), 1.0, jnp.float32)
    vzero = jnp.zeros((_L,), jnp.float32)
    viota = lax.iota(jnp.int32, _L)

    @plsc.parallel_loop(0, _CHV, 1, unroll=8)
    def _(v):
        onesb[pl.ds(v * _L, _L)] = vone

    vo = onesb[pl.ds(0, _L)]
    zb[pl.ds(0, _L)] = vzero
    vz = zb[pl.ds(0, _L)]

    @plsc.parallel_loop(0, 9 * _CHV, 1, unroll=8)
    def _(v):
        pb0[pl.ds(v * _L, _L)] = vzero
        pb1[pl.ds(v * _L, _L)] = vzero

    def _in_copy(xb, si, r, off):
        return pltpu.make_async_copy(x_hbm.at[r, pl.ds(off, _CH)], xb, si)

    def _out_copies(pbk, so, r, off):
        cps = [pltpu.make_async_copy(pbk.at[pl.ds(c * _CH, _CH)],
                                     out_hbm.at[c + 1, r, pl.ds(off, _CH)],
                                     so)
               for c in range(9)]
        cps.append(pltpu.make_async_copy(onesb, out_hbm.at[0, r, pl.ds(off, _CH)], so))
        return cps

    _in_copy(xb0, si0, rbase, 0).start()

    def _bucket(xv):
        acc = jnp.where(xv > thr[0], 1, 0)
        for th in thr[1:]:
            acc = acc + jnp.where(xv > th, 1, 0)
        return acc

    def _chunk(r, k, xb, pbk, ibk, si, so):
        off = k * _CH

        if k == 0:
            _in_copy(xb1, si1, r, _CH).start()
        else:
            @pl.when(r + 1 < rbase + rows_per_w)
            def _():
                _in_copy(xb0, si0, r + 1, 0).start()

        @pl.when(r > rbase)
        def _():
            for cp in _out_copies(pbk, so, r, off):
                cp.wait()

        _in_copy(xb, si, r, off).wait()

        @pl.when(r > rbase)
        def _():
            @plsc.parallel_loop(0, _CHV, 1, unroll=8)
            def _(v):
                old = ibk[pl.ds(v * _L, _L)]
                xv = xb[pl.ds(v * _L, _L)]
                new = _bucket(xv) * _CH + (v * _L + viota)
                ibk[pl.ds(v * _L, _L)] = new
                plsc.store_scatter(pbk, [new], vo)
                plsc.store_scatter(pbk, [old], vz, mask=old != new)

        @pl.when(r == rbase)
        def _():
            @plsc.parallel_loop(0, _CHV, 1, unroll=8)
            def _(v):
                xv = xb[pl.ds(v * _L, _L)]
                new = _bucket(xv) * _CH + (v * _L + viota)
                ibk[pl.ds(v * _L, _L)] = new
                plsc.store_scatter(pbk, [new], vo)

        for cp in _out_copies(pbk, so, r, off):
            cp.start()

    def _row(r, _):
        _chunk(r, 0, xb0, pb0, ib0, si0, so0)
        _chunk(r, 1, xb1, pb1, ib1, si1, so1)
        return 0
    lax.fori_loop(rbase, rbase + rows_per_w, _row, 0)

    last = rbase + rows_per_w - 1
    for cp in _out_copies(pb0, so0, last, 0):
        cp.wait()
    for cp in _out_copies(pb1, so1, last, _CH):
        cp.wait()


def kernel(x, bins):
    M, N = x.shape
    mesh = plsc.VectorSubcoreMesh(core_axis_name="c", subcore_axis_name="s")
    run = pl.kernel(
        _sc_body,
        out_type=jax.ShapeDtypeStruct((_D, M, N), jnp.float32),
        mesh=mesh,
        scratch_types=[
            pltpu.VMEM((_CH,), jnp.float32),
            pltpu.VMEM((_CH,), jnp.float32),
            pltpu.VMEM((9 * _CH,), jnp.float32),
            pltpu.VMEM((9 * _CH,), jnp.float32),
            pltpu.VMEM((_CH,), jnp.int32),
            pltpu.VMEM((_CH,), jnp.int32),
            pltpu.VMEM((_CH,), jnp.float32),
            pltpu.VMEM((16,), jnp.float32),
            pltpu.VMEM((16,), jnp.float32),
            pltpu.SemaphoreType.DMA,
            pltpu.SemaphoreType.DMA,
            pltpu.SemaphoreType.DMA,
            pltpu.SemaphoreType.DMA,
        ],
        compiler_params=pltpu.CompilerParams(needs_layout_passes=False),
    )
    out = run(x, bins)
    return jnp.transpose(out, (1, 2, 0))
